# Initial kernel scaffold; baseline (speedup 1.0000x reference)
import functools

import jax
import jax.numpy as jnp
from jax import lax
from jax.experimental import pallas as pl
from jax.experimental.pallas import tpu as pltpu
from jax.experimental.pallas import tpu_sc as plsc

N = 2048
E = 32768
H = 1024
IN_CH = 128
HID = 64
LAT = 32

NC = 2
NS = 16
NW = NC * NS
EPW = E // NW
CH = 128
NCH = EPW // CH
RPT = N // NS


_RB = 256


def _ka_body(my_ref, a_ref, cs_ref):
    i = pl.program_id(1)
    v = my_ref[...]
    a = jnp.where(v != 0.0, jax.nn.sigmoid(v), 0.0)
    a_ref[...] = a
    part = jnp.sum(a, axis=0, keepdims=True)

    @pl.when(i == 0)
    def _():
        cs_ref[...] = part

    @pl.when(i != 0)
    def _():
        cs_ref[...] = cs_ref[...] + part


def _make_a(my):
    return pl.pallas_call(
        _ka_body,
        grid=(2, H // _RB),
        in_specs=[pl.BlockSpec((_RB, H), lambda b, i: (b * (H // _RB) + i, b))],
        out_specs=[
            pl.BlockSpec((_RB, H), lambda b, i: (b * (H // _RB) + i, 0)),
            pl.BlockSpec((1, H), lambda b, i: (b, 0)),
        ],
        out_shape=[
            jax.ShapeDtypeStruct((N, H), jnp.float32),
            jax.ShapeDtypeStruct((2, H), jnp.float32),
        ],
    )(my)


_sc_mesh = plsc.VectorSubcoreMesh(core_axis_name="c", subcore_axis_name="s")


@functools.partial(
    pl.kernel,
    out_type=jax.ShapeDtypeStruct((NC, N), jnp.float32),
    mesh=_sc_mesh,
    scratch_types=[
        pltpu.VMEM((NCH, CH), jnp.int32),
        pltpu.VMEM((CH,), jnp.float32),
        pltpu.VMEM((RPT,), jnp.float32),
        pltpu.VMEM_SHARED((N,), jnp.float32),
    ],
)
def _hist_kernel(col3_hbm, out_hbm, idx_v, ones_v, half_v, hist_sh):
    cid = lax.axis_index("c")
    sid = lax.axis_index("s")
    wid = sid * NC + cid
    for k in range(CH // 16):
        ones_v[pl.ds(k * 16, 16)] = jnp.full((16,), 1.0, jnp.float32)
    for k in range(RPT // 16):
        half_v[pl.ds(k * 16, 16)] = jnp.full((16,), 0.5, jnp.float32)
    pltpu.sync_copy(half_v, hist_sh.at[pl.ds(sid * RPT, RPT)])
    pltpu.sync_copy(col3_hbm.at[wid], idx_v)
    plsc.subcore_barrier()
    for j in range(NCH):
        pltpu.sync_copy(ones_v, hist_sh.at[idx_v.at[j]], add=True)
    plsc.subcore_barrier()
    pltpu.sync_copy(
        hist_sh.at[pl.ds(sid * RPT, RPT)], out_hbm.at[cid, pl.ds(sid * RPT, RPT)]
    )


def _ks1_body(cs_ref, h0_ref, h1_ref, x_ref, w1_ref, dis_ref, s1_ref, s1h_ref):
    deg = cs_ref[...] + h0_ref[...] + h1_ref[...]
    dis = lax.rsqrt(deg)
    dis_ref[...] = dis
    xw = jnp.dot(x_ref[...], w1_ref[...], preferred_element_type=jnp.float32)
    s = dis * xw
    s1_ref[...] = s
    s1h_ref[...] = 0.5 * s


def _make_s1(cs_col, h0_col, h1_col, x, w1):
    return pl.pallas_call(
        _ks1_body,
        out_shape=[
            jax.ShapeDtypeStruct((N, 1), jnp.float32),
            jax.ShapeDtypeStruct((N, HID), jnp.float32),
            jax.ShapeDtypeStruct((N, HID), jnp.float32),
        ],
    )(cs_col, h0_col, h1_col, x, w1)


def _scat_body(s_hbm, sh_hbm, row3_hbm, col3_hbm, out_hbm,
               ridx_v, cidx_v, rows_v, u_sh, sem):
    cid = lax.axis_index("c")
    sid = lax.axis_index("s")
    wid = sid * NC + cid
    pltpu.sync_copy(sh_hbm.at[pl.ds(sid * RPT, RPT)],
                    u_sh.at[pl.ds(sid * RPT, RPT)])
    pltpu.sync_copy(row3_hbm.at[wid], ridx_v)
    pltpu.sync_copy(col3_hbm.at[wid], cidx_v)
    plsc.subcore_barrier()
    for j in range(NCH):
        pltpu.async_copy(s_hbm.at[ridx_v.at[j]], rows_v, sem).wait()
        pltpu.sync_copy(rows_v, u_sh.at[cidx_v.at[j]], add=True)
    plsc.subcore_barrier()
    pltpu.sync_copy(u_sh.at[pl.ds(sid * RPT, RPT)],
                    out_hbm.at[cid, pl.ds(sid * RPT, RPT)])


_scatter = pl.kernel(
    _scat_body,
    out_type=jax.ShapeDtypeStruct((NC, N, HID), jnp.float32),
    mesh=_sc_mesh,
    scratch_types=[
        pltpu.VMEM((NCH, CH), jnp.int32),
        pltpu.VMEM((NCH, CH), jnp.int32),
        pltpu.VMEM((CH, HID), jnp.float32),
        pltpu.VMEM_SHARED((N, HID), jnp.float32),
        pltpu.SemaphoreType.DMA,
    ],
)


def _kc1_body(a_ref, s1_ref, u_ref, dis_ref, b1_ref, wcat_ref, s2_ref, s2h_ref):
    t = lax.dot_general(a_ref[...], s1_ref[...], (((0,), (0,)), ((), ())),
                        preferred_element_type=jnp.float32)
    u = u_ref[0] + u_ref[1]
    pre = dis_ref[...] * (t + u) + b1_ref[...]
    hid = jnp.maximum(pre, 0.0)
    s2 = dis_ref[...] * jnp.dot(hid, wcat_ref[...],
                                preferred_element_type=jnp.float32)
    s2_ref[...] = s2
    s2h_ref[...] = 0.5 * s2


def _make_s2(a, s1, u1, dis, b1r, wcat):
    return pl.pallas_call(
        _kc1_body,
        grid=(2,),
        in_specs=[
            pl.BlockSpec((H, H), lambda b: (b, 0)),
            pl.BlockSpec((H, HID), lambda b: (b, 0)),
            pl.BlockSpec((NC, H, HID), lambda b: (0, b, 0)),
            pl.BlockSpec((H, 1), lambda b: (b, 0)),
            pl.BlockSpec((1, HID), lambda b: (0, 0)),
            pl.BlockSpec((HID, HID), lambda b: (0, 0)),
        ],
        out_specs=[
            pl.BlockSpec((H, HID), lambda b: (b, 0)),
            pl.BlockSpec((H, HID), lambda b: (b, 0)),
        ],
        out_shape=[
            jax.ShapeDtypeStruct((N, HID), jnp.float32),
            jax.ShapeDtypeStruct((N, HID), jnp.float32),
        ],
    )(a, s1, u1, dis, b1r, wcat)


def _ko_body(a_ref, s2_ref, u_ref, dis_ref, bcat_ref, mu_ref, ls_ref):
    t = lax.dot_general(a_ref[...], s2_ref[...], (((0,), (0,)), ((), ())),
                        preferred_element_type=jnp.float32)
    u = u_ref[0] + u_ref[1]
    o = dis_ref[...] * (t + u) + bcat_ref[...]
    mu_ref[...] = o[:, :LAT]
    ls_ref[...] = o[:, LAT:]


def _make_out(a, s2, u2, dis, bcat):
    return pl.pallas_call(
        _ko_body,
        grid=(2,),
        in_specs=[
            pl.BlockSpec((H, H), lambda b: (b, 0)),
            pl.BlockSpec((H, HID), lambda b: (b, 0)),
            pl.BlockSpec((NC, H, HID), lambda b: (0, b, 0)),
            pl.BlockSpec((H, 1), lambda b: (b, 0)),
            pl.BlockSpec((1, HID), lambda b: (0, 0)),
        ],
        out_specs=[
            pl.BlockSpec((H, LAT), lambda b: (b, 0)),
            pl.BlockSpec((H, LAT), lambda b: (b, 0)),
        ],
        out_shape=[
            jax.ShapeDtypeStruct((N, LAT), jnp.float32),
            jax.ShapeDtypeStruct((N, LAT), jnp.float32),
        ],
    )(a, s2, u2, dis, bcat)


def kernel(x, edge_index, masked_y, W1, b1, W_mu, b_mu, W_logstd, b_logstd):
    row3 = edge_index[0].reshape(NW, NCH, CH).astype(jnp.int32)
    col3 = edge_index[1].reshape(NW, NCH, CH).astype(jnp.int32)
    wcat = jnp.concatenate([W_mu, W_logstd], axis=1)
    bcat = jnp.concatenate([b_mu, b_logstd]).reshape(1, HID)
    b1r = b1.reshape(1, HID)

    a, cs = _make_a(masked_y)
    histp = _hist_kernel(col3)
    cs_col = cs.reshape(N, 1)
    h0_col = histp[0].reshape(N, 1)
    h1_col = histp[1].reshape(N, 1)
    dis, s1, s1h = _make_s1(cs_col, h0_col, h1_col, x, W1)
    u1 = _scatter(s1, s1h, row3, col3)
    s2, s2h = _make_s2(a, s1, u1, dis, b1r, wcat)
    u2 = _scatter(s2, s2h, row3, col3)
    z_mu, z_logstd = _make_out(a, s2, u2, dis, bcat)
    return (z_mu, z_logstd)


# speedup vs baseline: 811.5662x; 811.5662x over previous
import functools

import jax
import jax.numpy as jnp
from jax import lax
from jax.experimental import pallas as pl
from jax.experimental.pallas import tpu as pltpu
from jax.experimental.pallas import tpu_sc as plsc

N = 2048
E = 32768
H = 1024
IN_CH = 128
HID = 64
LAT = 32

NC = 2
NS = 16
NW = NC * NS
EPW = E // NW
CH = 128
NCH = EPW // CH
RPT = N // NS


_RB = 256


def _ka_body(my_ref, a_ref, cs_ref):
    i = pl.program_id(0)
    v = my_ref[...]
    a = jnp.where(v != 0.0, jax.nn.sigmoid(v), 0.0)
    a_ref[...] = a
    part = jnp.sum(a, axis=0, keepdims=True)

    @pl.when(i == 0)
    def _():
        cs_ref[...] = part

    @pl.when(i != 0)
    def _():
        cs_ref[...] = cs_ref[...] + part


def _make_a(my):
    return pl.pallas_call(
        _ka_body,
        grid=(H // _RB,),
        in_specs=[pl.BlockSpec((_RB, H), lambda i: (i, 0))],
        out_specs=[
            pl.BlockSpec((_RB, H), lambda i: (i, 0)),
            pl.BlockSpec((1, H), lambda i: (0, 0)),
        ],
        out_shape=[
            jax.ShapeDtypeStruct((H, H), jnp.float32),
            jax.ShapeDtypeStruct((1, H), jnp.float32),
        ],
    )(my)


_sc_mesh = plsc.VectorSubcoreMesh(core_axis_name="c", subcore_axis_name="s")
_sc_params = pltpu.CompilerParams(use_tc_tiling_on_sc=False)


@functools.partial(
    pl.kernel,
    out_type=jax.ShapeDtypeStruct((NC, N), jnp.float32),
    mesh=_sc_mesh,
    scratch_types=[
        pltpu.VMEM((NCH, CH), jnp.int32),
        pltpu.VMEM((CH,), jnp.float32),
        pltpu.VMEM((RPT,), jnp.float32),
        pltpu.VMEM_SHARED((N,), jnp.float32),
    ],
    compiler_params=_sc_params,
)
def _hist_kernel(col3_hbm, out_hbm, idx_v, ones_v, half_v, hist_sh):
    cid = lax.axis_index("c")
    sid = lax.axis_index("s")
    wid = sid * NC + cid
    for k in range(CH // 16):
        ones_v[pl.ds(k * 16, 16)] = jnp.full((16,), 1.0, jnp.float32)
    for k in range(RPT // 16):
        half_v[pl.ds(k * 16, 16)] = jnp.full((16,), 0.5, jnp.float32)
    pltpu.sync_copy(half_v, hist_sh.at[pl.ds(sid * RPT, RPT)])
    pltpu.sync_copy(col3_hbm.at[wid], idx_v)
    plsc.subcore_barrier()
    for j in range(NCH):
        pltpu.sync_copy(ones_v, hist_sh.at[idx_v.at[j]], add=True)
    plsc.subcore_barrier()
    pltpu.sync_copy(
        hist_sh.at[pl.ds(sid * RPT, RPT)], out_hbm.at[cid, pl.ds(sid * RPT, RPT)]
    )


def _ks1_body(cs_ref, h0_ref, h1_ref, x_ref, w1_ref, dis_ref, s1_ref, s1h_ref):
    deg = cs_ref[...] + h0_ref[...] + h1_ref[...]
    dis = lax.rsqrt(deg)
    dis_ref[...] = dis
    xw = jnp.dot(x_ref[...], w1_ref[...], preferred_element_type=jnp.float32)
    s = dis * xw
    s1_ref[...] = s
    s1h_ref[...] = 0.5 * s


def _make_s1(cs_col, h0_col, h1_col, x, w1):
    return pl.pallas_call(
        _ks1_body,
        out_shape=[
            jax.ShapeDtypeStruct((N, 1), jnp.float32),
            jax.ShapeDtypeStruct((N, HID), jnp.float32),
            jax.ShapeDtypeStruct((N, HID), jnp.float32),
        ],
    )(cs_col, h0_col, h1_col, x, w1)


def _scat_body(s_hbm, sh_hbm, row3_hbm, col3_hbm, out_hbm,
               ridx_v, cidx_v, rows_v, u_sh, sem):
    cid = lax.axis_index("c")
    sid = lax.axis_index("s")
    wid = sid * NC + cid
    pltpu.sync_copy(sh_hbm.at[pl.ds(sid * RPT, RPT)],
                    u_sh.at[pl.ds(sid * RPT, RPT)])
    pltpu.sync_copy(row3_hbm.at[wid], ridx_v)
    pltpu.sync_copy(col3_hbm.at[wid], cidx_v)
    plsc.subcore_barrier()
    for j in range(NCH):
        pltpu.async_copy(s_hbm.at[ridx_v.at[j]], rows_v, sem).wait()
        pltpu.sync_copy(rows_v, u_sh.at[cidx_v.at[j]], add=True)
    plsc.subcore_barrier()
    pltpu.sync_copy(u_sh.at[pl.ds(sid * RPT, RPT)],
                    out_hbm.at[cid, pl.ds(sid * RPT, RPT)])


_scatter = pl.kernel(
    _scat_body,
    out_type=jax.ShapeDtypeStruct((NC, N, HID), jnp.float32),
    mesh=_sc_mesh,
    scratch_types=[
        pltpu.VMEM((NCH, CH), jnp.int32),
        pltpu.VMEM((NCH, CH), jnp.int32),
        pltpu.VMEM((CH, HID), jnp.float32),
        pltpu.VMEM_SHARED((N, HID), jnp.float32),
        pltpu.SemaphoreType.DMA,
    ],
    compiler_params=_sc_params,
)


def _kc1_body(a_ref, s1_ref, u_ref, dis_ref, b1_ref, wcat_ref, s2_ref, s2h_ref):
    t_top = lax.dot_general(a_ref[...], s1_ref[:H, :], (((0,), (0,)), ((), ())),
                            preferred_element_type=jnp.float32)
    u = u_ref[0] + u_ref[1]
    pre_top = dis_ref[:H, :] * (t_top + u[:H, :]) + b1_ref[...]
    pre_bot = dis_ref[H:, :] * u[H:, :] + b1_ref[...]
    hid_top = jnp.maximum(pre_top, 0.0)
    hid_bot = jnp.maximum(pre_bot, 0.0)
    s2_top = dis_ref[:H, :] * jnp.dot(hid_top, wcat_ref[...],
                                      preferred_element_type=jnp.float32)
    s2_bot = dis_ref[H:, :] * jnp.dot(hid_bot, wcat_ref[...],
                                      preferred_element_type=jnp.float32)
    s2_ref[:H, :] = s2_top
    s2_ref[H:, :] = s2_bot
    s2h_ref[:H, :] = 0.5 * s2_top
    s2h_ref[H:, :] = 0.5 * s2_bot


def _make_s2(a, s1, u1, dis, b1r, wcat):
    return pl.pallas_call(
        _kc1_body,
        out_shape=[
            jax.ShapeDtypeStruct((N, HID), jnp.float32),
            jax.ShapeDtypeStruct((N, HID), jnp.float32),
        ],
    )(a, s1, u1, dis, b1r, wcat)


def _ko_body(a_ref, s2_ref, u_ref, dis_ref, bcat_ref, mu_ref, ls_ref):
    t_top = lax.dot_general(a_ref[...], s2_ref[:H, :], (((0,), (0,)), ((), ())),
                            preferred_element_type=jnp.float32)
    u = u_ref[0] + u_ref[1]
    o_top = dis_ref[:H, :] * (t_top + u[:H, :]) + bcat_ref[...]
    o_bot = dis_ref[H:, :] * u[H:, :] + bcat_ref[...]
    mu_ref[:H, :] = o_top[:, :LAT]
    mu_ref[H:, :] = o_bot[:, :LAT]
    ls_ref[:H, :] = o_top[:, LAT:]
    ls_ref[H:, :] = o_bot[:, LAT:]


def _make_out(a, s2, u2, dis, bcat):
    return pl.pallas_call(
        _ko_body,
        out_shape=[
            jax.ShapeDtypeStruct((N, LAT), jnp.float32),
            jax.ShapeDtypeStruct((N, LAT), jnp.float32),
        ],
    )(a, s2, u2, dis, bcat)


def kernel(x, edge_index, masked_y, W1, b1, W_mu, b_mu, W_logstd, b_logstd):
    row3 = edge_index[0].reshape(NW, NCH, CH).astype(jnp.int32)
    col3 = edge_index[1].reshape(NW, NCH, CH).astype(jnp.int32)
    wcat = jnp.concatenate([W_mu, W_logstd], axis=1)
    bcat = jnp.concatenate([b_mu, b_logstd]).reshape(1, HID)
    b1r = b1.reshape(1, HID)
    my_top = masked_y[:H, :H]

    a, cs = _make_a(my_top)
    histp = _hist_kernel(col3)
    cs_col = jnp.pad(cs.reshape(H), (0, N - H)).reshape(N, 1)
    h0_col = histp[0].reshape(N, 1)
    h1_col = histp[1].reshape(N, 1)
    dis, s1, s1h = _make_s1(cs_col, h0_col, h1_col, x, W1)
    u1 = _scatter(s1, s1h, row3, col3)
    s2, s2h = _make_s2(a, s1, u1, dis, b1r, wcat)
    u2 = _scatter(s2, s2h, row3, col3)
    z_mu, z_logstd = _make_out(a, s2, u2, dis, bcat)
    return (z_mu, z_logstd)


# speedup vs baseline: 937.0759x; 1.1547x over previous
import functools

import jax
import jax.numpy as jnp
from jax import lax
from jax.experimental import pallas as pl
from jax.experimental.pallas import tpu as pltpu
from jax.experimental.pallas import tpu_sc as plsc

N = 2048
E = 32768
H = 1024
IN_CH = 128
HID = 64
LAT = 32

NC = 2
NS = 16
NW = NC * NS
EPW = E // NW
CH = 128
NCH = EPW // CH
RPT = N // NS

_sc_mesh = plsc.VectorSubcoreMesh(core_axis_name="c", subcore_axis_name="s")
_sc_params = pltpu.CompilerParams(use_tc_tiling_on_sc=False)


@functools.partial(
    pl.kernel,
    out_type=jax.ShapeDtypeStruct((NC, N), jnp.float32),
    mesh=_sc_mesh,
    scratch_types=[
        pltpu.VMEM((NCH, CH), jnp.int32),
        pltpu.VMEM((CH,), jnp.float32),
        pltpu.VMEM((RPT,), jnp.float32),
        pltpu.VMEM_SHARED((N,), jnp.float32),
        pltpu.SemaphoreType.DMA,
    ],
    compiler_params=_sc_params,
)
def _hist_kernel(ei_hbm, out_hbm, idx_v, ones_v, half_v, hist_sh, sem):
    cid = lax.axis_index("c")
    sid = lax.axis_index("s")
    wid = sid * NC + cid
    base = wid * EPW
    descs = []
    for j in range(NCH):
        descs.append(
            pltpu.async_copy(ei_hbm.at[1, pl.ds(base + j * CH, CH)],
                             idx_v.at[j], sem))
    for k in range(CH // 16):
        ones_v[pl.ds(k * 16, 16)] = jnp.full((16,), 1.0, jnp.float32)
    for k in range(RPT // 16):
        half_v[pl.ds(k * 16, 16)] = jnp.full((16,), 0.5, jnp.float32)
    pltpu.sync_copy(half_v, hist_sh.at[pl.ds(sid * RPT, RPT)])
    for d in descs:
        d.wait()
    plsc.subcore_barrier()
    for j in range(NCH):
        pltpu.sync_copy(ones_v, hist_sh.at[idx_v.at[j]], add=True)
    plsc.subcore_barrier()
    pltpu.sync_copy(
        hist_sh.at[pl.ds(sid * RPT, RPT)], out_hbm.at[cid, pl.ds(sid * RPT, RPT)]
    )


_RB = 256
_NSTEPS = H // _RB


def _prep_body(my_ref, h_ref, x_ref, w1_ref,
               a_ref, dis_ref, s1_ref, s1h_ref, cs_ref):
    i = pl.program_id(0)
    v = my_ref[...]
    a = jnp.where(v != 0.0, jax.nn.sigmoid(v), 0.0)
    a_ref[...] = a
    ones_rb = jnp.ones((_RB, 1), jnp.float32)
    part = lax.dot_general(a, ones_rb, (((0,), (0,)), ((), ())),
                           preferred_element_type=jnp.float32)

    @pl.when(i == 0)
    def _():
        cs_ref[...] = part

    @pl.when(i != 0)
    def _():
        cs_ref[...] = cs_ref[...] + part

    @pl.when(i == _NSTEPS - 1)
    def _():
        ones2 = jnp.ones((2, 1), jnp.float32)
        h_col = lax.dot_general(h_ref[...], ones2, (((0,), (0,)), ((), ())),
                                preferred_element_type=jnp.float32)
        deg_top = cs_ref[...] + h_col[:H, :]
        deg_bot = h_col[H:, :]
        dis = lax.rsqrt(jnp.concatenate([deg_top, deg_bot], axis=0))
        dis_ref[...] = dis
        xw = jnp.dot(x_ref[...], w1_ref[...],
                     preferred_element_type=jnp.float32)
        s = dis * xw
        s1_ref[...] = s
        s1h_ref[...] = 0.5 * s


def _prep(my, histp, x, w1):
    return pl.pallas_call(
        _prep_body,
        grid=(_NSTEPS,),
        in_specs=[
            pl.BlockSpec((_RB, H), lambda i: (i, 0)),
            pl.BlockSpec((NC, N), lambda i: (0, 0)),
            pl.BlockSpec((N, IN_CH), lambda i: (0, 0)),
            pl.BlockSpec((IN_CH, HID), lambda i: (0, 0)),
        ],
        out_specs=[
            pl.BlockSpec((_RB, H), lambda i: (i, 0)),
            pl.BlockSpec((N, 1), lambda i: (0, 0)),
            pl.BlockSpec((N, HID), lambda i: (0, 0)),
            pl.BlockSpec((N, HID), lambda i: (0, 0)),
        ],
        out_shape=[
            jax.ShapeDtypeStruct((H, H), jnp.float32),
            jax.ShapeDtypeStruct((N, 1), jnp.float32),
            jax.ShapeDtypeStruct((N, HID), jnp.float32),
            jax.ShapeDtypeStruct((N, HID), jnp.float32),
        ],
        scratch_shapes=[pltpu.VMEM((H, 1), jnp.float32)],
    )(my, histp, x, w1)


def _scat_body(s_hbm, sh_hbm, ei_hbm, out_hbm,
               ridx_v, cidx_v, rows_v, u_sh, sem_i, sem_ld, sem_g):
    cid = lax.axis_index("c")
    sid = lax.axis_index("s")
    wid = sid * NC + cid
    base = wid * EPW
    d_init = pltpu.async_copy(sh_hbm.at[pl.ds(sid * RPT, RPT)],
                              u_sh.at[pl.ds(sid * RPT, RPT)], sem_i)
    descs = []
    for j in range(NCH):
        descs.append(
            pltpu.async_copy(ei_hbm.at[0, pl.ds(base + j * CH, CH)],
                             ridx_v.at[j], sem_ld))
        descs.append(
            pltpu.async_copy(ei_hbm.at[1, pl.ds(base + j * CH, CH)],
                             cidx_v.at[j], sem_ld))
    for d in descs:
        d.wait()
    g = pltpu.async_copy(s_hbm.at[ridx_v.at[0]], rows_v.at[0], sem_g)
    d_init.wait()
    plsc.subcore_barrier()
    for j in range(NCH):
        g.wait()
        if j + 1 < NCH:
            g = pltpu.async_copy(s_hbm.at[ridx_v.at[j + 1]],
                                 rows_v.at[(j + 1) % 2], sem_g)
        pltpu.sync_copy(rows_v.at[j % 2], u_sh.at[cidx_v.at[j]], add=True)
    plsc.subcore_barrier()
    pltpu.sync_copy(u_sh.at[pl.ds(sid * RPT, RPT)],
                    out_hbm.at[cid, pl.ds(sid * RPT, RPT)])


_scatter = pl.kernel(
    _scat_body,
    out_type=jax.ShapeDtypeStruct((NC, N, HID), jnp.float32),
    mesh=_sc_mesh,
    scratch_types=[
        pltpu.VMEM((NCH, CH), jnp.int32),
        pltpu.VMEM((NCH, CH), jnp.int32),
        pltpu.VMEM((2, CH, HID), jnp.float32),
        pltpu.VMEM_SHARED((N, HID), jnp.float32),
        pltpu.SemaphoreType.DMA,
        pltpu.SemaphoreType.DMA,
        pltpu.SemaphoreType.DMA,
    ],
    compiler_params=_sc_params,
)


def _kc1_body(a_ref, s1_ref, u_ref, dis_ref, b1_ref, wmu_ref, wls_ref,
              s2_ref, s2h_ref):
    t_top = lax.dot_general(a_ref[...], s1_ref[:H, :], (((0,), (0,)), ((), ())),
                            preferred_element_type=jnp.float32)
    u = u_ref[0] + u_ref[1]
    b1v = b1_ref[...]
    pre_top = dis_ref[:H, :] * (t_top + u[:H, :]) + b1v
    pre_bot = dis_ref[H:, :] * u[H:, :] + b1v
    hid_top = jnp.maximum(pre_top, 0.0)
    hid_bot = jnp.maximum(pre_bot, 0.0)
    wc = jnp.concatenate([wmu_ref[...], wls_ref[...]], axis=1)
    s2_top = dis_ref[:H, :] * jnp.dot(hid_top, wc,
                                      preferred_element_type=jnp.float32)
    s2_bot = dis_ref[H:, :] * jnp.dot(hid_bot, wc,
                                      preferred_element_type=jnp.float32)
    s2_ref[:H, :] = s2_top
    s2_ref[H:, :] = s2_bot
    s2h_ref[:H, :] = 0.5 * s2_top
    s2h_ref[H:, :] = 0.5 * s2_bot


def _make_s2(a, s1, u1, dis, b1, wmu, wls):
    return pl.pallas_call(
        _kc1_body,
        out_shape=[
            jax.ShapeDtypeStruct((N, HID), jnp.float32),
            jax.ShapeDtypeStruct((N, HID), jnp.float32),
        ],
    )(a, s1, u1, dis, b1, wmu, wls)


def _ko_body(a_ref, s2_ref, u_ref, dis_ref, bmu_ref, bls_ref, mu_ref, ls_ref):
    t_top = lax.dot_general(a_ref[...], s2_ref[:H, :], (((0,), (0,)), ((), ())),
                            preferred_element_type=jnp.float32)
    u = u_ref[0] + u_ref[1]
    o_top = dis_ref[:H, :] * (t_top + u[:H, :])
    o_bot = dis_ref[H:, :] * u[H:, :]
    bmu = bmu_ref[...]
    bls = bls_ref[...]
    mu_ref[:H, :] = o_top[:, :LAT] + bmu
    mu_ref[H:, :] = o_bot[:, :LAT] + bmu
    ls_ref[:H, :] = o_top[:, LAT:] + bls
    ls_ref[H:, :] = o_bot[:, LAT:] + bls


def _make_out(a, s2, u2, dis, bmu, bls):
    return pl.pallas_call(
        _ko_body,
        out_shape=[
            jax.ShapeDtypeStruct((N, LAT), jnp.float32),
            jax.ShapeDtypeStruct((N, LAT), jnp.float32),
        ],
    )(a, s2, u2, dis, bmu, bls)


def kernel(x, edge_index, masked_y, W1, b1, W_mu, b_mu, W_logstd, b_logstd):
    ei = edge_index.astype(jnp.int32)
    histp = _hist_kernel(ei)
    a, dis, s1, s1h = _prep(masked_y, histp, x, W1)
    u1 = _scatter(s1, s1h, ei)
    s2, s2h = _make_s2(a, s1, u1, dis, b1, W_mu, W_logstd)
    u2 = _scatter(s2, s2h, ei)
    z_mu, z_logstd = _make_out(a, s2, u2, dis, b_mu, b_logstd)
    return (z_mu, z_logstd)


# speedup vs baseline: 1001.1048x; 1.0683x over previous
import functools

import jax
import jax.numpy as jnp
from jax import lax
from jax.experimental import pallas as pl
from jax.experimental.pallas import tpu as pltpu
from jax.experimental.pallas import tpu_sc as plsc

N = 2048
E = 32768
H = 1024
IN_CH = 128
HID = 64
HID2 = 2 * HID
LAT = 32

NC = 2
NS = 16
NW = NC * NS
EPW = E // NW
CH = 128
NCH = EPW // CH
RPT = N // NS

_sc_mesh = plsc.VectorSubcoreMesh(core_axis_name="c", subcore_axis_name="s")
_sc_params = pltpu.CompilerParams(use_tc_tiling_on_sc=True)


@functools.partial(
    pl.kernel,
    out_type=jax.ShapeDtypeStruct((NC, N), jnp.float32),
    mesh=_sc_mesh,
    scratch_types=[
        pltpu.VMEM((NCH, CH), jnp.int32),
        pltpu.VMEM((CH,), jnp.float32),
        pltpu.VMEM((RPT,), jnp.float32),
        pltpu.VMEM_SHARED((N,), jnp.float32),
        pltpu.SemaphoreType.DMA,
    ],
    compiler_params=_sc_params,
)
def _hist_kernel(col_hbm, out_hbm, idx_v, ones_v, half_v, hist_sh, sem):
    cid = lax.axis_index("c")
    sid = lax.axis_index("s")
    wid = sid * NC + cid
    base = wid * EPW
    descs = []
    for j in range(NCH):
        descs.append(
            pltpu.async_copy(col_hbm.at[pl.ds(base + j * CH, CH)],
                             idx_v.at[j], sem))
    for k in range(CH // 16):
        ones_v[pl.ds(k * 16, 16)] = jnp.full((16,), 1.0, jnp.float32)
    for k in range(RPT // 16):
        half_v[pl.ds(k * 16, 16)] = jnp.full((16,), 0.5, jnp.float32)
    pltpu.sync_copy(half_v, hist_sh.at[pl.ds(sid * RPT, RPT)])
    for d in descs:
        d.wait()
    plsc.subcore_barrier()
    for j in range(NCH):
        pltpu.sync_copy(ones_v, hist_sh.at[idx_v.at[j]], add=True)
    plsc.subcore_barrier()
    pltpu.sync_copy(
        hist_sh.at[pl.ds(sid * RPT, RPT)], out_hbm.at[cid, pl.ds(sid * RPT, RPT)]
    )


_RB = 256
_NSTEPS = H // _RB


def _prep_body(my_ref, h_ref, x_ref, w1_ref,
               a_ref, dis_ref, s1_ref, cs_ref):
    i = pl.program_id(0)
    v = my_ref[...]
    a = jnp.where(v != 0.0, jax.nn.sigmoid(v), 0.0)
    a_ref[...] = a
    ones_rb = jnp.ones((_RB, 1), jnp.float32)
    part = lax.dot_general(a, ones_rb, (((0,), (0,)), ((), ())),
                           preferred_element_type=jnp.float32)

    @pl.when(i == 0)
    def _():
        cs_ref[...] = part

    @pl.when(i != 0)
    def _():
        cs_ref[...] = cs_ref[...] + part

    @pl.when(i == _NSTEPS - 1)
    def _():
        ones2 = jnp.ones((2, 1), jnp.float32)
        h_col = lax.dot_general(h_ref[...], ones2, (((0,), (0,)), ((), ())),
                                preferred_element_type=jnp.float32)
        deg_top = cs_ref[...] + h_col[:H, :]
        deg_bot = h_col[H:, :]
        dis = lax.rsqrt(jnp.concatenate([deg_top, deg_bot], axis=0))
        dis_ref[...] = dis
        xw = jnp.dot(x_ref[...], w1_ref[...],
                     preferred_element_type=jnp.float32)
        s = dis * xw
        s1_ref[...] = jnp.concatenate([s, 0.5 * s], axis=1)


def _prep(my, histp, x, w1):
    return pl.pallas_call(
        _prep_body,
        grid=(_NSTEPS,),
        in_specs=[
            pl.BlockSpec((_RB, H), lambda i: (i, 0)),
            pl.BlockSpec((NC, N), lambda i: (0, 0)),
            pl.BlockSpec((N, IN_CH), lambda i: (0, 0)),
            pl.BlockSpec((IN_CH, HID), lambda i: (0, 0)),
        ],
        out_specs=[
            pl.BlockSpec((_RB, H), lambda i: (i, 0)),
            pl.BlockSpec((N, 1), lambda i: (0, 0)),
            pl.BlockSpec((N, HID2), lambda i: (0, 0)),
        ],
        out_shape=[
            jax.ShapeDtypeStruct((H, H), jnp.float32),
            jax.ShapeDtypeStruct((N, 1), jnp.float32),
            jax.ShapeDtypeStruct((N, HID2), jnp.float32),
        ],
        scratch_shapes=[pltpu.VMEM((H, 1), jnp.float32)],
    )(my, histp, x, w1)


def _scat_body(sp_hbm, row_hbm, col_hbm, out_hbm,
               ridx_v, cidx_v, rows_v, u_sh, sem_i, sem_ld, sem_g):
    cid = lax.axis_index("c")
    sid = lax.axis_index("s")
    wid = sid * NC + cid
    base = wid * EPW
    descs = []
    for j in range(NCH):
        descs.append(
            pltpu.async_copy(row_hbm.at[pl.ds(base + j * CH, CH)],
                             ridx_v.at[j], sem_ld))
        descs.append(
            pltpu.async_copy(col_hbm.at[pl.ds(base + j * CH, CH)],
                             cidx_v.at[j], sem_ld))
    d_init = pltpu.async_copy(sp_hbm.at[pl.ds(sid * RPT, RPT)],
                              u_sh.at[pl.ds(sid * RPT, RPT)], sem_i)
    for d in descs:
        d.wait()
    g = pltpu.async_copy(sp_hbm.at[ridx_v.at[0]], rows_v.at[0], sem_g)
    d_init.wait()
    plsc.subcore_barrier()
    for j in range(NCH):
        g.wait()
        if j + 1 < NCH:
            g = pltpu.async_copy(sp_hbm.at[ridx_v.at[j + 1]],
                                 rows_v.at[(j + 1) % 2], sem_g)
        pltpu.sync_copy(rows_v.at[j % 2], u_sh.at[cidx_v.at[j]], add=True)
    plsc.subcore_barrier()
    pltpu.sync_copy(u_sh.at[pl.ds(sid * RPT, RPT)],
                    out_hbm.at[cid, pl.ds(sid * RPT, RPT)])


_scatter = pl.kernel(
    _scat_body,
    out_type=jax.ShapeDtypeStruct((NC, N, HID2), jnp.float32),
    mesh=_sc_mesh,
    scratch_types=[
        pltpu.VMEM((NCH, CH), jnp.int32),
        pltpu.VMEM((NCH, CH), jnp.int32),
        pltpu.VMEM((2, CH, HID2), jnp.float32),
        pltpu.VMEM_SHARED((N, HID2), jnp.float32),
        pltpu.SemaphoreType.DMA,
        pltpu.SemaphoreType.DMA,
        pltpu.SemaphoreType.DMA,
    ],
    compiler_params=_sc_params,
)


def _kc1_body(a_ref, s1_ref, u_ref, dis_ref, b1_ref, wmu_ref, wls_ref,
              s2_ref):
    s1 = s1_ref[:, :HID]
    t_top = lax.dot_general(a_ref[...], s1[:H, :], (((0,), (0,)), ((), ())),
                            preferred_element_type=jnp.float32)
    u = u_ref[0, :, :HID] + u_ref[1, :, :HID] - s1
    b1v = b1_ref[...]
    pre_top = dis_ref[:H, :] * (t_top + u[:H, :]) + b1v
    pre_bot = dis_ref[H:, :] * u[H:, :] + b1v
    hid_top = jnp.maximum(pre_top, 0.0)
    hid_bot = jnp.maximum(pre_bot, 0.0)
    wc = jnp.concatenate([wmu_ref[...], wls_ref[...]], axis=1)
    s2_top = dis_ref[:H, :] * jnp.dot(hid_top, wc,
                                      preferred_element_type=jnp.float32)
    s2_bot = dis_ref[H:, :] * jnp.dot(hid_bot, wc,
                                      preferred_element_type=jnp.float32)
    s2_ref[:H, :] = jnp.concatenate([s2_top, 0.5 * s2_top], axis=1)
    s2_ref[H:, :] = jnp.concatenate([s2_bot, 0.5 * s2_bot], axis=1)


def _make_s2(a, s1, u1, dis, b1, wmu, wls):
    return pl.pallas_call(
        _kc1_body,
        out_shape=jax.ShapeDtypeStruct((N, HID2), jnp.float32),
    )(a, s1, u1, dis, b1, wmu, wls)


def _ko_body(a_ref, s2_ref, u_ref, dis_ref, bmu_ref, bls_ref, mu_ref, ls_ref):
    s2 = s2_ref[:, :HID]
    t_top = lax.dot_general(a_ref[...], s2[:H, :], (((0,), (0,)), ((), ())),
                            preferred_element_type=jnp.float32)
    u = u_ref[0, :, :HID] + u_ref[1, :, :HID] - s2
    o_top = dis_ref[:H, :] * (t_top + u[:H, :])
    o_bot = dis_ref[H:, :] * u[H:, :]
    bmu = bmu_ref[...]
    bls = bls_ref[...]
    mu_ref[:H, :] = o_top[:, :LAT] + bmu
    mu_ref[H:, :] = o_bot[:, :LAT] + bmu
    ls_ref[:H, :] = o_top[:, LAT:] + bls
    ls_ref[H:, :] = o_bot[:, LAT:] + bls


def _make_out(a, s2, u2, dis, bmu, bls):
    return pl.pallas_call(
        _ko_body,
        out_shape=[
            jax.ShapeDtypeStruct((N, LAT), jnp.float32),
            jax.ShapeDtypeStruct((N, LAT), jnp.float32),
        ],
    )(a, s2, u2, dis, bmu, bls)


def kernel(x, edge_index, masked_y, W1, b1, W_mu, b_mu, W_logstd, b_logstd):
    ei = edge_index.astype(jnp.int32)
    row = ei[0]
    col = ei[1]
    histp = _hist_kernel(col)
    a, dis, s1p = _prep(masked_y, histp, x, W1)
    u1 = _scatter(s1p, row, col)
    s2p = _make_s2(a, s1p, u1, dis, b1, W_mu, W_logstd)
    u2 = _scatter(s2p, row, col)
    z_mu, z_logstd = _make_out(a, s2p, u2, dis, b_mu, b_logstd)
    return (z_mu, z_logstd)


# speedup vs baseline: 1049.8859x; 1.0487x over previous
import functools

import jax
import jax.numpy as jnp
from jax import lax
from jax.experimental import pallas as pl
from jax.experimental.pallas import tpu as pltpu
from jax.experimental.pallas import tpu_sc as plsc

N = 2048
E = 32768
H = 1024
IN_CH = 128
HID = 64
HID2 = 2 * HID
LAT = 32

NC = 2
NS = 16
NW = NC * NS
EPW = E // NW
CH = 128
NCH = EPW // CH
RPT = N // NS

_sc_mesh = plsc.VectorSubcoreMesh(core_axis_name="c", subcore_axis_name="s")
_sc_params = pltpu.CompilerParams(use_tc_tiling_on_sc=True)


@functools.partial(
    pl.kernel,
    out_type=jax.ShapeDtypeStruct((NC, N), jnp.float32),
    mesh=_sc_mesh,
    scratch_types=[
        pltpu.VMEM((NCH, CH), jnp.int32),
        pltpu.VMEM((CH,), jnp.float32),
        pltpu.VMEM((RPT,), jnp.float32),
        pltpu.VMEM_SHARED((N,), jnp.float32),
        pltpu.SemaphoreType.DMA,
    ],
    compiler_params=_sc_params,
)
def _hist_kernel(col_hbm, out_hbm, idx_v, ones_v, half_v, hist_sh, sem):
    cid = lax.axis_index("c")
    sid = lax.axis_index("s")
    wid = sid * NC + cid
    base = wid * EPW
    descs = []
    for j in range(NCH):
        descs.append(
            pltpu.async_copy(col_hbm.at[pl.ds(base + j * CH, CH)],
                             idx_v.at[j], sem))
    for k in range(CH // 16):
        ones_v[pl.ds(k * 16, 16)] = jnp.full((16,), 1.0, jnp.float32)
    for k in range(RPT // 16):
        half_v[pl.ds(k * 16, 16)] = jnp.full((16,), 0.5, jnp.float32)
    pltpu.sync_copy(half_v, hist_sh.at[pl.ds(sid * RPT, RPT)])
    for d in descs:
        d.wait()
    plsc.subcore_barrier()
    for j in range(NCH):
        pltpu.sync_copy(ones_v, hist_sh.at[idx_v.at[j]], add=True)
    plsc.subcore_barrier()
    pltpu.sync_copy(
        hist_sh.at[pl.ds(sid * RPT, RPT)], out_hbm.at[cid, pl.ds(sid * RPT, RPT)]
    )


_RB = 256
_NSTEPS = H // _RB


def _ka_body(my_ref, a_ref, cs_ref):
    i = pl.program_id(0)
    v = my_ref[...]
    a = jnp.where(v != 0.0, jax.nn.sigmoid(v), 0.0)
    a_ref[...] = a
    ones_rb = jnp.ones((_RB, 1), jnp.float32)
    part = lax.dot_general(a, ones_rb, (((0,), (0,)), ((), ())),
                           preferred_element_type=jnp.float32)

    @pl.when(i == 0)
    def _():
        cs_ref[...] = part

    @pl.when(i != 0)
    def _():
        cs_ref[...] = cs_ref[...] + part


def _make_a(my):
    return pl.pallas_call(
        _ka_body,
        grid=(_NSTEPS,),
        in_specs=[pl.BlockSpec((_RB, H), lambda i: (i, 0))],
        out_specs=[
            pl.BlockSpec((_RB, H), lambda i: (i, 0)),
            pl.BlockSpec((H, 1), lambda i: (0, 0)),
        ],
        out_shape=[
            jax.ShapeDtypeStruct((H, H), jnp.float32),
            jax.ShapeDtypeStruct((H, 1), jnp.float32),
        ],
    )(my)


def _ks1_body(cs_ref, h_ref, x_ref, w1_ref, dis_ref, s1_ref):
    ones2 = jnp.ones((2, 1), jnp.float32)
    h_col = lax.dot_general(h_ref[...], ones2, (((0,), (0,)), ((), ())),
                            preferred_element_type=jnp.float32)
    deg_top = cs_ref[...] + h_col[:H, :]
    deg_bot = h_col[H:, :]
    dis = lax.rsqrt(jnp.concatenate([deg_top, deg_bot], axis=0))
    dis_ref[...] = dis
    xw = jnp.dot(x_ref[...], w1_ref[...], preferred_element_type=jnp.float32)
    s = dis * xw
    s1_ref[...] = jnp.concatenate([s, 0.5 * s], axis=1)


def _make_s1(cs, histp, x, w1):
    return pl.pallas_call(
        _ks1_body,
        out_shape=[
            jax.ShapeDtypeStruct((N, 1), jnp.float32),
            jax.ShapeDtypeStruct((N, HID2), jnp.float32),
        ],
    )(cs, histp, x, w1)


def _kt_body(a_ref, sp_ref, t_ref):
    t_ref[...] = lax.dot_general(a_ref[...], sp_ref[:H, :HID],
                                 (((0,), (0,)), ((), ())),
                                 preferred_element_type=jnp.float32)


def _make_t(a, sp):
    return pl.pallas_call(
        _kt_body,
        out_shape=jax.ShapeDtypeStruct((H, HID), jnp.float32),
    )(a, sp)


def _scat_body(sp_hbm, row_hbm, col_hbm, out_hbm,
               ridx_v, cidx_v, rows_v, u_sh, sem_i, sem_ld, sem_g):
    cid = lax.axis_index("c")
    sid = lax.axis_index("s")
    wid = sid * NC + cid
    base = wid * EPW
    descs = []
    for j in range(NCH):
        descs.append(
            pltpu.async_copy(row_hbm.at[pl.ds(base + j * CH, CH)],
                             ridx_v.at[j], sem_ld))
        descs.append(
            pltpu.async_copy(col_hbm.at[pl.ds(base + j * CH, CH)],
                             cidx_v.at[j], sem_ld))
    d_init = pltpu.async_copy(sp_hbm.at[pl.ds(sid * RPT, RPT)],
                              u_sh.at[pl.ds(sid * RPT, RPT)], sem_i)
    for d in descs:
        d.wait()
    g = pltpu.async_copy(sp_hbm.at[ridx_v.at[0]], rows_v.at[0], sem_g)
    d_init.wait()
    plsc.subcore_barrier()
    for j in range(NCH):
        g.wait()
        if j + 1 < NCH:
            g = pltpu.async_copy(sp_hbm.at[ridx_v.at[j + 1]],
                                 rows_v.at[(j + 1) % 2], sem_g)
        pltpu.sync_copy(rows_v.at[j % 2], u_sh.at[cidx_v.at[j]], add=True)
    plsc.subcore_barrier()
    pltpu.sync_copy(u_sh.at[pl.ds(sid * RPT, RPT)],
                    out_hbm.at[cid, pl.ds(sid * RPT, RPT)])


_scatter = pl.kernel(
    _scat_body,
    out_type=jax.ShapeDtypeStruct((NC, N, HID2), jnp.float32),
    mesh=_sc_mesh,
    scratch_types=[
        pltpu.VMEM((NCH, CH), jnp.int32),
        pltpu.VMEM((NCH, CH), jnp.int32),
        pltpu.VMEM((2, CH, HID2), jnp.float32),
        pltpu.VMEM_SHARED((N, HID2), jnp.float32),
        pltpu.SemaphoreType.DMA,
        pltpu.SemaphoreType.DMA,
        pltpu.SemaphoreType.DMA,
    ],
    compiler_params=_sc_params,
)


def _kc1_body(t_ref, s1_ref, u_ref, dis_ref, b1_ref, wmu_ref, wls_ref,
              s2_ref):
    s1 = s1_ref[:, :HID]
    t_top = t_ref[...]
    u = u_ref[0, :, :HID] + u_ref[1, :, :HID] - s1
    b1v = b1_ref[...]
    pre_top = dis_ref[:H, :] * (t_top + u[:H, :]) + b1v
    pre_bot = dis_ref[H:, :] * u[H:, :] + b1v
    hid_top = jnp.maximum(pre_top, 0.0)
    hid_bot = jnp.maximum(pre_bot, 0.0)
    wc = jnp.concatenate([wmu_ref[...], wls_ref[...]], axis=1)
    s2_top = dis_ref[:H, :] * jnp.dot(hid_top, wc,
                                      preferred_element_type=jnp.float32)
    s2_bot = dis_ref[H:, :] * jnp.dot(hid_bot, wc,
                                      preferred_element_type=jnp.float32)
    s2_ref[:H, :] = jnp.concatenate([s2_top, 0.5 * s2_top], axis=1)
    s2_ref[H:, :] = jnp.concatenate([s2_bot, 0.5 * s2_bot], axis=1)


def _make_s2(t1, s1, u1, dis, b1, wmu, wls):
    return pl.pallas_call(
        _kc1_body,
        out_shape=jax.ShapeDtypeStruct((N, HID2), jnp.float32),
    )(t1, s1, u1, dis, b1, wmu, wls)


def _ko_body(t_ref, s2_ref, u_ref, dis_ref, bmu_ref, bls_ref, mu_ref, ls_ref):
    s2 = s2_ref[:, :HID]
    t_top = t_ref[...]
    u = u_ref[0, :, :HID] + u_ref[1, :, :HID] - s2
    o_top = dis_ref[:H, :] * (t_top + u[:H, :])
    o_bot = dis_ref[H:, :] * u[H:, :]
    bmu = bmu_ref[...]
    bls = bls_ref[...]
    mu_ref[:H, :] = o_top[:, :LAT] + bmu
    mu_ref[H:, :] = o_bot[:, :LAT] + bmu
    ls_ref[:H, :] = o_top[:, LAT:] + bls
    ls_ref[H:, :] = o_bot[:, LAT:] + bls


def _make_out(t2, s2, u2, dis, bmu, bls):
    return pl.pallas_call(
        _ko_body,
        out_shape=[
            jax.ShapeDtypeStruct((N, LAT), jnp.float32),
            jax.ShapeDtypeStruct((N, LAT), jnp.float32),
        ],
    )(t2, s2, u2, dis, bmu, bls)


def kernel(x, edge_index, masked_y, W1, b1, W_mu, b_mu, W_logstd, b_logstd):
    ei = edge_index.astype(jnp.int32)
    row = ei[0]
    col = ei[1]
    histp = _hist_kernel(col)
    a, cs = _make_a(masked_y)
    dis, s1p = _make_s1(cs, histp, x, W1)
    u1 = _scatter(s1p, row, col)
    t1 = _make_t(a, s1p)
    s2p = _make_s2(t1, s1p, u1, dis, b1, W_mu, W_logstd)
    u2 = _scatter(s2p, row, col)
    t2 = _make_t(a, s2p)
    z_mu, z_logstd = _make_out(t2, s2p, u2, dis, b_mu, b_logstd)
    return (z_mu, z_logstd)


# speedup vs baseline: 1125.1465x; 1.0717x over previous
import functools

import jax
import jax.numpy as jnp
from jax import lax
from jax.experimental import pallas as pl
from jax.experimental.pallas import tpu as pltpu
from jax.experimental.pallas import tpu_sc as plsc

N = 2048
E = 32768
H = 1024
IN_CH = 128
HID = 64
HID2 = 2 * HID
LAT = 32

NC = 2
NS = 16
NW = NC * NS
EPW = E // NW
CH = 128
NCH = EPW // CH
RPT = N // NS

_sc_mesh = plsc.VectorSubcoreMesh(core_axis_name="c", subcore_axis_name="s")
_sc_params = pltpu.CompilerParams(use_tc_tiling_on_sc=True)


@functools.partial(
    pl.kernel,
    out_type=jax.ShapeDtypeStruct((NC, N), jnp.float32),
    mesh=_sc_mesh,
    scratch_types=[
        pltpu.VMEM((NCH, CH), jnp.int32),
        pltpu.VMEM((CH,), jnp.float32),
        pltpu.VMEM((RPT,), jnp.float32),
        pltpu.VMEM_SHARED((N,), jnp.float32),
        pltpu.SemaphoreType.DMA,
    ],
    compiler_params=_sc_params,
)
def _hist_kernel(col_hbm, out_hbm, idx_v, ones_v, half_v, hist_sh, sem):
    cid = lax.axis_index("c")
    sid = lax.axis_index("s")
    wid = sid * NC + cid
    base = wid * EPW
    descs = []
    for j in range(NCH):
        descs.append(
            pltpu.async_copy(col_hbm.at[pl.ds(base + j * CH, CH)],
                             idx_v.at[j], sem))
    for k in range(CH // 16):
        ones_v[pl.ds(k * 16, 16)] = jnp.full((16,), 1.0, jnp.float32)
    for k in range(RPT // 16):
        half_v[pl.ds(k * 16, 16)] = jnp.full((16,), 0.5, jnp.float32)
    pltpu.sync_copy(half_v, hist_sh.at[pl.ds(sid * RPT, RPT)])
    for d in descs:
        d.wait()
    plsc.subcore_barrier()
    for j in range(NCH):
        pltpu.sync_copy(ones_v, hist_sh.at[idx_v.at[j]], add=True)
    plsc.subcore_barrier()
    pltpu.sync_copy(
        hist_sh.at[pl.ds(sid * RPT, RPT)], out_hbm.at[cid, pl.ds(sid * RPT, RPT)]
    )


_RB = 256
_NSTEPS = H // _RB


def _ka_body(my_ref, a_ref, cs_ref, csr_ref):
    i = pl.program_id(0)
    v = my_ref[...]
    a = jnp.where(v != 0.0, jax.nn.sigmoid(v), 0.0)
    a_ref[...] = a
    ones_rb = jnp.ones((_RB, 1), jnp.float32)
    part = lax.dot_general(a, ones_rb, (((0,), (0,)), ((), ())),
                           preferred_element_type=jnp.float32)
    part_row = jnp.sum(a, axis=0, keepdims=True)

    @pl.when(i == 0)
    def _():
        cs_ref[...] = part
        csr_ref[...] = part_row

    @pl.when(i != 0)
    def _():
        cs_ref[...] = cs_ref[...] + part
        csr_ref[...] = csr_ref[...] + part_row


def _make_a(my):
    return pl.pallas_call(
        _ka_body,
        grid=(_NSTEPS,),
        in_specs=[pl.BlockSpec((_RB, H), lambda i: (i, 0))],
        out_specs=[
            pl.BlockSpec((_RB, H), lambda i: (i, 0)),
            pl.BlockSpec((H, 1), lambda i: (0, 0)),
            pl.BlockSpec((1, H), lambda i: (0, 0)),
        ],
        out_shape=[
            jax.ShapeDtypeStruct((H, H), jnp.float32),
            jax.ShapeDtypeStruct((H, 1), jnp.float32),
            jax.ShapeDtypeStruct((1, H), jnp.float32),
        ],
    )(my)


def _ks1_body(cs_ref, csr_ref, h_ref, x_ref, w1_ref, dis_ref, disr_ref, s1_ref):
    ones2 = jnp.ones((2, 1), jnp.float32)
    h_col = lax.dot_general(h_ref[...], ones2, (((0,), (0,)), ((), ())),
                            preferred_element_type=jnp.float32)
    deg_top = cs_ref[...] + h_col[:H, :]
    deg_bot = h_col[H:, :]
    dis = lax.rsqrt(jnp.concatenate([deg_top, deg_bot], axis=0))
    dis_ref[...] = dis
    h_row = jnp.sum(h_ref[...], axis=0, keepdims=True)
    deg_row = jnp.concatenate(
        [csr_ref[...], jnp.zeros((1, N - H), jnp.float32)], axis=1) + h_row
    disr_ref[...] = lax.rsqrt(deg_row)
    xw = jnp.dot(x_ref[...], w1_ref[...], preferred_element_type=jnp.float32)
    s = dis * xw
    s1_ref[...] = jnp.concatenate([s, 0.5 * s], axis=1)


def _make_s1(cs, csr, histp, x, w1):
    return pl.pallas_call(
        _ks1_body,
        out_shape=[
            jax.ShapeDtypeStruct((N, 1), jnp.float32),
            jax.ShapeDtypeStruct((1, N), jnp.float32),
            jax.ShapeDtypeStruct((N, HID2), jnp.float32),
        ],
    )(cs, csr, histp, x, w1)


def _kt_body(a_ref, sp_ref, t_ref):
    t_ref[...] = lax.dot_general(a_ref[...], sp_ref[:H, :HID],
                                 (((0,), (0,)), ((), ())),
                                 preferred_element_type=jnp.float32)


def _make_t(a, sp):
    return pl.pallas_call(
        _kt_body,
        out_shape=jax.ShapeDtypeStruct((H, HID), jnp.float32),
    )(a, sp)


def _ktt_body(a_ref, sp_ref, t_ref):
    t_ref[...] = lax.dot_general(sp_ref[:H, :HID], a_ref[...],
                                 (((0,), (0,)), ((), ())),
                                 preferred_element_type=jnp.float32)


def _make_tT(a, sp):
    return pl.pallas_call(
        _ktt_body,
        out_shape=jax.ShapeDtypeStruct((HID, H), jnp.float32),
    )(a, sp)


def _scat_body(sp_hbm, row_hbm, col_hbm, out_hbm,
               ridx_v, cidx_v, rows_v, u_sh, sem_i, sem_ld, sem_g):
    cid = lax.axis_index("c")
    sid = lax.axis_index("s")
    wid = sid * NC + cid
    base = wid * EPW
    descs = []
    for j in range(NCH):
        descs.append(
            pltpu.async_copy(row_hbm.at[pl.ds(base + j * CH, CH)],
                             ridx_v.at[j], sem_ld))
        descs.append(
            pltpu.async_copy(col_hbm.at[pl.ds(base + j * CH, CH)],
                             cidx_v.at[j], sem_ld))
    d_init = pltpu.async_copy(sp_hbm.at[pl.ds(sid * RPT, RPT)],
                              u_sh.at[pl.ds(sid * RPT, RPT)], sem_i)
    for d in descs:
        d.wait()
    g = pltpu.async_copy(sp_hbm.at[ridx_v.at[0]], rows_v.at[0], sem_g)
    d_init.wait()
    plsc.subcore_barrier()
    for j in range(NCH):
        g.wait()
        if j + 1 < NCH:
            g = pltpu.async_copy(sp_hbm.at[ridx_v.at[j + 1]],
                                 rows_v.at[(j + 1) % 2], sem_g)
        pltpu.sync_copy(rows_v.at[j % 2], u_sh.at[cidx_v.at[j]], add=True)
    plsc.subcore_barrier()
    pltpu.sync_copy(u_sh.at[pl.ds(sid * RPT, RPT)],
                    out_hbm.at[cid, pl.ds(sid * RPT, RPT)])


_scatter = pl.kernel(
    _scat_body,
    out_type=jax.ShapeDtypeStruct((NC, N, HID2), jnp.float32),
    mesh=_sc_mesh,
    scratch_types=[
        pltpu.VMEM((NCH, CH), jnp.int32),
        pltpu.VMEM((NCH, CH), jnp.int32),
        pltpu.VMEM((2, CH, HID2), jnp.float32),
        pltpu.VMEM_SHARED((N, HID2), jnp.float32),
        pltpu.SemaphoreType.DMA,
        pltpu.SemaphoreType.DMA,
        pltpu.SemaphoreType.DMA,
    ],
    compiler_params=_sc_params,
)


def _kc1_body(t_ref, s1_ref, u_ref, dis_ref, b1_ref, wmu_ref, wls_ref,
              s2_ref):
    s1 = s1_ref[:, :HID]
    t_top = t_ref[...]
    u = u_ref[0, :, :HID] + u_ref[1, :, :HID] - s1
    b1v = b1_ref[...]
    pre_top = dis_ref[:H, :] * (t_top + u[:H, :]) + b1v
    pre_bot = dis_ref[H:, :] * u[H:, :] + b1v
    hid_top = jnp.maximum(pre_top, 0.0)
    hid_bot = jnp.maximum(pre_bot, 0.0)
    wc = jnp.concatenate([wmu_ref[...], wls_ref[...]], axis=1)
    s2_top = dis_ref[:H, :] * jnp.dot(hid_top, wc,
                                      preferred_element_type=jnp.float32)
    s2_bot = dis_ref[H:, :] * jnp.dot(hid_bot, wc,
                                      preferred_element_type=jnp.float32)
    s2_ref[:H, :] = jnp.concatenate([s2_top, 0.5 * s2_top], axis=1)
    s2_ref[H:, :] = jnp.concatenate([s2_bot, 0.5 * s2_bot], axis=1)


def _make_s2(t1, s1, u1, dis, b1, wmu, wls):
    return pl.pallas_call(
        _kc1_body,
        out_shape=jax.ShapeDtypeStruct((N, HID2), jnp.float32),
    )(t1, s1, u1, dis, b1, wmu, wls)


def _ko_body(tt_ref, s2_ref, u_ref, disr_ref, bmu_ref, bls_ref,
             mu_ref, ls_ref):
    s2 = s2_ref[:, :HID]
    u = u_ref[0, :, :HID] + u_ref[1, :, :HID] - s2
    ut = lax.transpose(u, (1, 0))
    left = tt_ref[...] + ut[:, :H]
    ot = disr_ref[...] * jnp.concatenate([left, ut[:, H:]], axis=1)
    mu_ref[...] = ot[:LAT, :] + bmu_ref[...]
    ls_ref[...] = ot[LAT:, :] + bls_ref[...]


def _make_out(t2t, s2, u2, disr, bmu, bls):
    return pl.pallas_call(
        _ko_body,
        out_shape=[
            jax.ShapeDtypeStruct((LAT, N), jnp.float32),
            jax.ShapeDtypeStruct((LAT, N), jnp.float32),
        ],
    )(t2t, s2, u2, disr, bmu, bls)


def kernel(x, edge_index, masked_y, W1, b1, W_mu, b_mu, W_logstd, b_logstd):
    ei = edge_index.astype(jnp.int32)
    row = ei[0]
    col = ei[1]
    histp = _hist_kernel(col)
    a, cs, csr = _make_a(masked_y)
    dis, disr, s1p = _make_s1(cs, csr, histp, x, W1)
    u1 = _scatter(s1p, row, col)
    t1 = _make_t(a, s1p)
    s2p = _make_s2(t1, s1p, u1, dis, b1, W_mu, W_logstd)
    u2 = _scatter(s2p, row, col)
    t2t = _make_tT(a, s2p)
    z_mu_t, z_logstd_t = _make_out(t2t, s2p, u2, disr,
                                   b_mu.reshape(LAT, 1), b_logstd.reshape(LAT, 1))
    return (z_mu_t.T, z_logstd_t.T)


# speedup vs baseline: 1202.9213x; 1.0691x over previous
import functools

import jax
import jax.numpy as jnp
from jax import lax
from jax.experimental import pallas as pl
from jax.experimental.pallas import tpu as pltpu
from jax.experimental.pallas import tpu_sc as plsc

N = 2048
E = 32768
H = 1024
IN_CH = 128
HID = 64
HID2 = 2 * HID
LAT = 32

NC = 2
NS = 16
NW = NC * NS
EPW = E // NW
CH = 128
NCH = EPW // CH
RPT = N // NS

_sc_mesh = plsc.VectorSubcoreMesh(core_axis_name="c", subcore_axis_name="s")
_sc_params = pltpu.CompilerParams(use_tc_tiling_on_sc=True)


@functools.partial(
    pl.kernel,
    out_type=jax.ShapeDtypeStruct((NC, N), jnp.float32),
    mesh=_sc_mesh,
    scratch_types=[
        pltpu.VMEM((NCH, CH), jnp.int32),
        pltpu.VMEM((CH,), jnp.float32),
        pltpu.VMEM((RPT,), jnp.float32),
        pltpu.VMEM_SHARED((N,), jnp.float32),
        pltpu.SemaphoreType.DMA,
    ],
    compiler_params=_sc_params,
)
def _hist_kernel(col_hbm, out_hbm, idx_v, ones_v, half_v, hist_sh, sem):
    cid = lax.axis_index("c")
    sid = lax.axis_index("s")
    wid = sid * NC + cid
    base = wid * EPW
    descs = []
    for j in range(NCH):
        descs.append(
            pltpu.async_copy(col_hbm.at[pl.ds(base + j * CH, CH)],
                             idx_v.at[j], sem))
    for k in range(CH // 16):
        ones_v[pl.ds(k * 16, 16)] = jnp.full((16,), 1.0, jnp.float32)
    for k in range(RPT // 16):
        half_v[pl.ds(k * 16, 16)] = jnp.full((16,), 0.5, jnp.float32)
    pltpu.sync_copy(half_v, hist_sh.at[pl.ds(sid * RPT, RPT)])
    for d in descs:
        d.wait()
    plsc.subcore_barrier()
    for j in range(NCH):
        pltpu.sync_copy(ones_v, hist_sh.at[idx_v.at[j]], add=True)
    plsc.subcore_barrier()
    pltpu.sync_copy(
        hist_sh.at[pl.ds(sid * RPT, RPT)], out_hbm.at[cid, pl.ds(sid * RPT, RPT)]
    )


_RB = 256
_NSTEPS = H // _RB


def _ka_body(my_ref, a_ref, cs_ref, csr_ref):
    i = pl.program_id(0)
    v = my_ref[...]
    a = jnp.where(v != 0.0, jax.nn.sigmoid(v), 0.0)
    a_ref[...] = a
    ones_rb = jnp.ones((_RB, 1), jnp.float32)
    part = lax.dot_general(a, ones_rb, (((0,), (0,)), ((), ())),
                           preferred_element_type=jnp.float32)
    part_row = jnp.sum(a, axis=0, keepdims=True)

    @pl.when(i == 0)
    def _():
        cs_ref[...] = part
        csr_ref[...] = part_row

    @pl.when(i != 0)
    def _():
        cs_ref[...] = cs_ref[...] + part
        csr_ref[...] = csr_ref[...] + part_row


def _make_a(my):
    return pl.pallas_call(
        _ka_body,
        grid=(_NSTEPS,),
        in_specs=[pl.BlockSpec((_RB, H), lambda i: (i, 0))],
        out_specs=[
            pl.BlockSpec((_RB, H), lambda i: (i, 0)),
            pl.BlockSpec((H, 1), lambda i: (0, 0)),
            pl.BlockSpec((1, H), lambda i: (0, 0)),
        ],
        out_shape=[
            jax.ShapeDtypeStruct((H, H), jnp.float32),
            jax.ShapeDtypeStruct((H, 1), jnp.float32),
            jax.ShapeDtypeStruct((1, H), jnp.float32),
        ],
    )(my)


def _ks1_body(cs_ref, csr_ref, h_ref, x_ref, w1_ref, dis_ref, disr_ref, s1_ref):
    ones2 = jnp.ones((2, 1), jnp.float32)
    h_col = lax.dot_general(h_ref[...], ones2, (((0,), (0,)), ((), ())),
                            preferred_element_type=jnp.float32)
    deg_top = cs_ref[...] + h_col[:H, :]
    deg_bot = h_col[H:, :]
    dis = lax.rsqrt(jnp.concatenate([deg_top, deg_bot], axis=0))
    dis_ref[...] = dis
    h_row = jnp.sum(h_ref[...], axis=0, keepdims=True)
    deg_row = jnp.concatenate(
        [csr_ref[...], jnp.zeros((1, N - H), jnp.float32)], axis=1) + h_row
    disr_ref[...] = lax.rsqrt(deg_row)
    xw = jnp.dot(x_ref[...], w1_ref[...], preferred_element_type=jnp.float32)
    s = dis * xw
    s1_ref[...] = jnp.concatenate([s, 0.5 * s], axis=1)


def _make_s1(cs, csr, histp, x, w1):
    return pl.pallas_call(
        _ks1_body,
        out_shape=[
            jax.ShapeDtypeStruct((N, 1), jnp.float32),
            jax.ShapeDtypeStruct((1, N), jnp.float32),
            jax.ShapeDtypeStruct((N, HID2), jnp.float32),
        ],
    )(cs, csr, histp, x, w1)


def _kt_body(a_ref, sp_ref, t_ref):
    t_ref[...] = lax.dot_general(a_ref[...], sp_ref[:H, :HID],
                                 (((0,), (0,)), ((), ())),
                                 preferred_element_type=jnp.float32)


def _make_t(a, sp):
    return pl.pallas_call(
        _kt_body,
        out_shape=jax.ShapeDtypeStruct((H, HID), jnp.float32),
    )(a, sp)


def _ktt_body(a_ref, sp_ref, t_ref):
    t_ref[...] = lax.dot_general(sp_ref[:H, :HID], a_ref[...],
                                 (((0,), (0,)), ((), ())),
                                 preferred_element_type=jnp.float32)


def _make_tT(a, sp):
    return pl.pallas_call(
        _ktt_body,
        out_shape=jax.ShapeDtypeStruct((HID, H), jnp.float32),
    )(a, sp)


def _scat_body(sp_hbm, row_hbm, col_hbm, out_hbm,
               ridx_v, cidx_v, rows_v, u_sh, sem_i, sem_ld, sem_g):
    cid = lax.axis_index("c")
    sid = lax.axis_index("s")
    wid = sid * NC + cid
    base = wid * EPW
    descs = []
    for j in range(NCH):
        descs.append(
            pltpu.async_copy(row_hbm.at[pl.ds(base + j * CH, CH)],
                             ridx_v.at[j], sem_ld))
        descs.append(
            pltpu.async_copy(col_hbm.at[pl.ds(base + j * CH, CH)],
                             cidx_v.at[j], sem_ld))
    d_init = pltpu.async_copy(sp_hbm.at[pl.ds(sid * RPT, RPT)],
                              u_sh.at[pl.ds(sid * RPT, RPT)], sem_i)
    for d in descs:
        d.wait()
    gs = [pltpu.async_copy(sp_hbm.at[ridx_v.at[j]], rows_v.at[j % _BUF], sem_g)
          for j in range(_BUF - 1)]
    d_init.wait()
    plsc.subcore_barrier()
    for j in range(NCH):
        gs[j].wait()
        if j + _BUF - 1 < NCH:
            gs.append(
                pltpu.async_copy(sp_hbm.at[ridx_v.at[j + _BUF - 1]],
                                 rows_v.at[(j + _BUF - 1) % _BUF], sem_g))
        pltpu.sync_copy(rows_v.at[j % _BUF], u_sh.at[cidx_v.at[j]], add=True)
    plsc.subcore_barrier()
    pltpu.sync_copy(u_sh.at[pl.ds(sid * RPT, RPT)],
                    out_hbm.at[cid, pl.ds(sid * RPT, RPT)])


_BUF = 3

_scatter = pl.kernel(
    _scat_body,
    out_type=jax.ShapeDtypeStruct((NC, N, HID2), jnp.float32),
    mesh=_sc_mesh,
    scratch_types=[
        pltpu.VMEM((NCH, CH), jnp.int32),
        pltpu.VMEM((NCH, CH), jnp.int32),
        pltpu.VMEM((_BUF, CH, HID2), jnp.float32),
        pltpu.VMEM_SHARED((N, HID2), jnp.float32),
        pltpu.SemaphoreType.DMA,
        pltpu.SemaphoreType.DMA,
        pltpu.SemaphoreType.DMA,
    ],
    compiler_params=_sc_params,
)


def _kc1_body(t_ref, s1_ref, u_ref, dis_ref, b1_ref, wmu_ref, wls_ref,
              s2_ref):
    s1 = s1_ref[:, :HID]
    t_top = t_ref[...]
    u = u_ref[0, :, :HID] + u_ref[1, :, :HID] - s1
    b1v = b1_ref[...]
    pre_top = dis_ref[:H, :] * (t_top + u[:H, :]) + b1v
    pre_bot = dis_ref[H:, :] * u[H:, :] + b1v
    hid_top = jnp.maximum(pre_top, 0.0)
    hid_bot = jnp.maximum(pre_bot, 0.0)
    wc = jnp.concatenate([wmu_ref[...], wls_ref[...]], axis=1)
    s2_top = dis_ref[:H, :] * jnp.dot(hid_top, wc,
                                      preferred_element_type=jnp.float32)
    s2_bot = dis_ref[H:, :] * jnp.dot(hid_bot, wc,
                                      preferred_element_type=jnp.float32)
    s2_ref[:H, :] = jnp.concatenate([s2_top, 0.5 * s2_top], axis=1)
    s2_ref[H:, :] = jnp.concatenate([s2_bot, 0.5 * s2_bot], axis=1)


def _make_s2(t1, s1, u1, dis, b1, wmu, wls):
    return pl.pallas_call(
        _kc1_body,
        out_shape=jax.ShapeDtypeStruct((N, HID2), jnp.float32),
    )(t1, s1, u1, dis, b1, wmu, wls)


def _ko_body(tt_ref, s2_ref, u_ref, disr_ref, bmu_ref, bls_ref,
             mu_ref, ls_ref):
    s2 = s2_ref[:, :HID]
    u = u_ref[0, :, :HID] + u_ref[1, :, :HID] - s2
    ut = lax.transpose(u, (1, 0))
    left = tt_ref[...] + ut[:, :H]
    ot = disr_ref[...] * jnp.concatenate([left, ut[:, H:]], axis=1)
    mu_ref[...] = ot[:LAT, :] + bmu_ref[...]
    ls_ref[...] = ot[LAT:, :] + bls_ref[...]


def _make_out(t2t, s2, u2, disr, bmu, bls):
    return pl.pallas_call(
        _ko_body,
        out_shape=[
            jax.ShapeDtypeStruct((LAT, N), jnp.float32),
            jax.ShapeDtypeStruct((LAT, N), jnp.float32),
        ],
    )(t2t, s2, u2, disr, bmu, bls)


def kernel(x, edge_index, masked_y, W1, b1, W_mu, b_mu, W_logstd, b_logstd):
    ei = edge_index.astype(jnp.int32)
    row = ei[0]
    col = ei[1]
    histp = _hist_kernel(col)
    a, cs, csr = _make_a(masked_y)
    dis, disr, s1p = _make_s1(cs, csr, histp, x, W1)
    u1 = _scatter(s1p, row, col)
    t1 = _make_t(a, s1p)
    s2p = _make_s2(t1, s1p, u1, dis, b1, W_mu, W_logstd)
    u2 = _scatter(s2p, row, col)
    t2t = _make_tT(a, s2p)
    z_mu_t, z_logstd_t = _make_out(t2t, s2p, u2, disr,
                                   b_mu.reshape(LAT, 1), b_logstd.reshape(LAT, 1))
    return (z_mu_t.T, z_logstd_t.T)
